# Initial kernel scaffold; baseline (speedup 1.0000x reference)
import jax
import jax.numpy as jnp
from jax.experimental import pallas as pl


def kernel(x, adj, w, a):
    raise NotImplementedError("write your pallas kernel here")


# speedup vs baseline: 1.5420x; 1.5420x over previous
import jax
import jax.numpy as jnp
from jax.experimental import pallas as pl

_ALPHA = 0.2


def _prologue_kernel(x_ref, w_ref, a_ref, h_ref, p_ref, r_ref, q_ref, t_ref):
    d = w_ref.shape[1]
    h = jnp.dot(x_ref[...], w_ref[...], preferred_element_type=jnp.float32)
    h_ref[...] = h
    f = jnp.sum(h * a_ref[0:1, :d], axis=1, keepdims=True)
    g = jnp.sum(h * a_ref[0:1, d:], axis=1, keepdims=True)
    p_ref[...] = jnp.exp(-f)
    r_ref[...] = jnp.exp(-_ALPHA * f)
    q_ref[...] = jnp.exp(-g)
    t_ref[...] = jnp.exp(-_ALPHA * g)


def _main_kernel(adj_ref, p_ref, r_ref, qt_ref, tt_ref, h_ref, out_ref):
    e = adj_ref[...] * jnp.minimum(p_ref[...] * qt_ref[...],
                                   r_ref[...] * tt_ref[...])
    acc = jnp.dot(e, h_ref[...], preferred_element_type=jnp.float32)
    rs = jnp.sum(e, axis=1, keepdims=True)
    hp = acc / rs
    out_ref[...] = jnp.where(hp > 0, hp, jnp.exp(hp) - 1.0)


def kernel(x, adj, w, a):
    n, d_in = x.shape
    d = w.shape[1]

    h, p, r, q, t = pl.pallas_call(
        _prologue_kernel,
        out_shape=(
            jax.ShapeDtypeStruct((n, d), jnp.float32),
            jax.ShapeDtypeStruct((n, 1), jnp.float32),
            jax.ShapeDtypeStruct((n, 1), jnp.float32),
            jax.ShapeDtypeStruct((n, 1), jnp.float32),
            jax.ShapeDtypeStruct((n, 1), jnp.float32),
        ),
    )(x, w, a)

    qt = q.reshape(1, n)
    tt = t.reshape(1, n)

    bi = 200 if n % 200 == 0 else n
    ni = n // bi

    out = pl.pallas_call(
        _main_kernel,
        grid=(ni,),
        in_specs=[
            pl.BlockSpec((bi, n), lambda i: (i, 0)),
            pl.BlockSpec((bi, 1), lambda i: (i, 0)),
            pl.BlockSpec((bi, 1), lambda i: (i, 0)),
            pl.BlockSpec((1, n), lambda i: (0, 0)),
            pl.BlockSpec((1, n), lambda i: (0, 0)),
            pl.BlockSpec((n, d), lambda i: (0, 0)),
        ],
        out_specs=pl.BlockSpec((bi, d), lambda i: (i, 0)),
        out_shape=jax.ShapeDtypeStruct((n, d), jnp.float32),
    )(adj, p, r, qt, tt, h)
    return out


# speedup vs baseline: 1.8272x; 1.1849x over previous
import jax
import jax.numpy as jnp
from jax.experimental import pallas as pl

_ALPHA = 0.2


def _prologue_kernel(x_ref, w_ref, a_ref, hb_ref, p_ref, r_ref,
                     q_ref, t_ref):
    d = w_ref.shape[1]
    h = jnp.dot(x_ref[...], w_ref[...], preferred_element_type=jnp.float32)
    hb_ref[:, :] = jnp.zeros_like(hb_ref)
    hb_ref[:, :d] = h.astype(jnp.bfloat16)
    hb_ref[:, d:d + 1] = jnp.ones((h.shape[0], 1), jnp.bfloat16)
    f = jnp.sum(h * a_ref[0:1, :d], axis=1, keepdims=True)
    g = jnp.sum(h * a_ref[0:1, d:], axis=1, keepdims=True)
    p_ref[...] = jnp.exp(-f)
    r_ref[...] = jnp.exp(-_ALPHA * f)
    q_ref[...] = jnp.exp(-g)
    t_ref[...] = jnp.exp(-_ALPHA * g)


def _main_kernel(adj_ref, p_ref, r_ref, qt_ref, tt_ref, hb_ref, out_ref):
    d = out_ref.shape[1]
    e = (adj_ref[...] * jnp.minimum(p_ref[...] * qt_ref[...],
                                    r_ref[...] * tt_ref[...])
         ).astype(jnp.bfloat16)
    acc = jnp.dot(e, hb_ref[...], preferred_element_type=jnp.float32)
    hp = acc[:, :d] / acc[:, d:d + 1]
    out_ref[...] = jnp.where(hp > 0, hp, jnp.exp(hp) - 1.0)


def kernel(x, adj, w, a):
    n, d_in = x.shape
    d = w.shape[1]

    hb, p, r, q, t = pl.pallas_call(
        _prologue_kernel,
        out_shape=(
            jax.ShapeDtypeStruct((n, 2 * d), jnp.bfloat16),
            jax.ShapeDtypeStruct((n, 1), jnp.float32),
            jax.ShapeDtypeStruct((n, 1), jnp.float32),
            jax.ShapeDtypeStruct((n, 1), jnp.float32),
            jax.ShapeDtypeStruct((n, 1), jnp.float32),
        ),
    )(x, w, a)

    qt = q.reshape(1, n)
    tt = t.reshape(1, n)

    bi = 200 if n % 200 == 0 else n
    ni = n // bi

    out = pl.pallas_call(
        _main_kernel,
        grid=(ni,),
        in_specs=[
            pl.BlockSpec((bi, n), lambda i: (i, 0)),
            pl.BlockSpec((bi, 1), lambda i: (i, 0)),
            pl.BlockSpec((bi, 1), lambda i: (i, 0)),
            pl.BlockSpec((1, n), lambda i: (0, 0)),
            pl.BlockSpec((1, n), lambda i: (0, 0)),
            pl.BlockSpec((n, 2 * d), lambda i: (0, 0)),
        ],
        out_specs=pl.BlockSpec((bi, d), lambda i: (i, 0)),
        out_shape=jax.ShapeDtypeStruct((n, d), jnp.float32),
    )(adj, p, r, qt, tt, hb)
    return out


# speedup vs baseline: 1.8276x; 1.0003x over previous
import jax
import jax.numpy as jnp
from jax.experimental import pallas as pl
from jax.experimental.pallas import tpu as pltpu

_ALPHA = 0.2


def _prologue_kernel(x_ref, w_ref, a_ref, hb_ref, p_ref, r_ref,
                     q_ref, t_ref):
    d = w_ref.shape[1]
    h = jnp.dot(x_ref[...], w_ref[...], preferred_element_type=jnp.float32)
    hb_ref[:, :] = jnp.zeros_like(hb_ref)
    hb_ref[:, :d] = h.astype(jnp.bfloat16)
    hb_ref[:, d:d + 1] = jnp.ones((h.shape[0], 1), jnp.bfloat16)
    f = jnp.sum(h * a_ref[0:1, :d], axis=1, keepdims=True)
    g = jnp.sum(h * a_ref[0:1, d:], axis=1, keepdims=True)
    p_ref[...] = jnp.exp(-f)
    r_ref[...] = jnp.exp(-_ALPHA * f)
    q_ref[...] = jnp.exp(-g)
    t_ref[...] = jnp.exp(-_ALPHA * g)


def _main_kernel(adj_ref, p_ref, r_ref, qt_ref, tt_ref, hb_ref, out_ref):
    d = out_ref.shape[1]
    e = (adj_ref[...] * jnp.minimum(p_ref[...] * qt_ref[...],
                                    r_ref[...] * tt_ref[...])
         ).astype(jnp.bfloat16)
    acc = jnp.dot(e, hb_ref[...], preferred_element_type=jnp.float32)
    hp = acc[:, :d] / acc[:, d:d + 1]
    out_ref[...] = jnp.where(hp > 0, hp, jnp.exp(hp) - 1.0)


def kernel(x, adj, w, a):
    n, d_in = x.shape
    d = w.shape[1]

    hb, p, r, q, t = pl.pallas_call(
        _prologue_kernel,
        out_shape=(
            jax.ShapeDtypeStruct((n, 2 * d), jnp.bfloat16),
            jax.ShapeDtypeStruct((n, 1), jnp.float32),
            jax.ShapeDtypeStruct((n, 1), jnp.float32),
            jax.ShapeDtypeStruct((n, 1), jnp.float32),
            jax.ShapeDtypeStruct((n, 1), jnp.float32),
        ),
    )(x, w, a)

    qt = q.reshape(1, n)
    tt = t.reshape(1, n)

    bi = 200 if n % 200 == 0 else n
    ni = n // bi

    out = pl.pallas_call(
        _main_kernel,
        grid=(ni,),
        in_specs=[
            pl.BlockSpec((bi, n), lambda i: (i, 0)),
            pl.BlockSpec((bi, 1), lambda i: (i, 0)),
            pl.BlockSpec((bi, 1), lambda i: (i, 0)),
            pl.BlockSpec((1, n), lambda i: (0, 0)),
            pl.BlockSpec((1, n), lambda i: (0, 0)),
            pl.BlockSpec((n, 2 * d), lambda i: (0, 0)),
        ],
        out_specs=pl.BlockSpec((bi, d), lambda i: (i, 0)),
        out_shape=jax.ShapeDtypeStruct((n, d), jnp.float32),
        compiler_params=pltpu.CompilerParams(
            dimension_semantics=("parallel",)),
    )(adj, p, r, qt, tt, hb)
    return out


# speedup vs baseline: 1.8718x; 1.0242x over previous
import jax
import jax.numpy as jnp
from jax.experimental import pallas as pl
from jax.experimental.pallas import tpu as pltpu

_ALPHA = 0.2


def _prologue_kernel(x_ref, w_ref, a_ref, hb_ref, p_ref, r_ref,
                     q_ref, t_ref):
    d = w_ref.shape[1]
    h = jnp.dot(x_ref[...], w_ref[...], preferred_element_type=jnp.float32)
    hb_ref[:, :] = jnp.zeros_like(hb_ref)
    hb_ref[:, :d] = h.astype(jnp.bfloat16)
    hb_ref[:, d:d + 1] = jnp.ones((h.shape[0], 1), jnp.bfloat16)
    f = jnp.sum(h * a_ref[0:1, :d], axis=1, keepdims=True)
    g = jnp.sum(h * a_ref[0:1, d:], axis=1, keepdims=True)
    p_ref[...] = jnp.exp(-f)
    r_ref[...] = jnp.exp(-_ALPHA * f)
    q_ref[...] = jnp.exp(-g)
    t_ref[...] = jnp.exp(-_ALPHA * g)


def _main_kernel(adj_ref, p_ref, r_ref, qt_ref, tt_ref, hb_ref, out_ref):
    d = out_ref.shape[1]
    e = (adj_ref[...] * jnp.minimum(p_ref[...] * qt_ref[...],
                                    r_ref[...] * tt_ref[...])
         ).astype(jnp.bfloat16)
    acc = jnp.dot(e, hb_ref[...], preferred_element_type=jnp.float32)
    hp = acc[:, :d] / acc[:, d:d + 1]
    out_ref[...] = jnp.where(hp > 0, hp, jnp.exp(hp) - 1.0)


def kernel(x, adj, w, a):
    n, d_in = x.shape
    d = w.shape[1]

    hb, p, r, q, t = pl.pallas_call(
        _prologue_kernel,
        out_shape=(
            jax.ShapeDtypeStruct((n, 2 * d), jnp.bfloat16),
            jax.ShapeDtypeStruct((n, 1), jnp.float32),
            jax.ShapeDtypeStruct((n, 1), jnp.float32),
            jax.ShapeDtypeStruct((n, 1), jnp.float32),
            jax.ShapeDtypeStruct((n, 1), jnp.float32),
        ),
    )(x, w, a)

    qt = q.reshape(1, n)
    tt = t.reshape(1, n)

    bi = 400 if n % 400 == 0 else n
    ni = n // bi

    out = pl.pallas_call(
        _main_kernel,
        grid=(ni,),
        in_specs=[
            pl.BlockSpec((bi, n), lambda i: (i, 0)),
            pl.BlockSpec((bi, 1), lambda i: (i, 0)),
            pl.BlockSpec((bi, 1), lambda i: (i, 0)),
            pl.BlockSpec((1, n), lambda i: (0, 0)),
            pl.BlockSpec((1, n), lambda i: (0, 0)),
            pl.BlockSpec((n, 2 * d), lambda i: (0, 0)),
        ],
        out_specs=pl.BlockSpec((bi, d), lambda i: (i, 0)),
        out_shape=jax.ShapeDtypeStruct((n, d), jnp.float32),
        compiler_params=pltpu.CompilerParams(
            dimension_semantics=("parallel",)),
    )(adj, p, r, qt, tt, hb)
    return out
